# baseline (speedup 1.0000x reference)
import jax
import jax.numpy as jnp
from jax import lax
from jax.experimental import pallas as pl
from jax.experimental.pallas import tpu as pltpu
from jax.experimental.pallas import tpu_sc as plsc

B = 1024
S = 50
NW = 32
NB_W = B // NW
LANES = 16
ROWS_B = S * S
REL_CHUNKS = ((0, 1248), (1248, 1248), (2496, 1248), (3744, 1256))
CMAX = 1256
J_CHUNKS = (0, 16, 32, 34)


def _splat(v):
    return jnp.full((LANES,), v, jnp.int32)


def _body(x_hbm, ppost_hbm, ppara_hbm, dpost_hbm, dpara_hbm,
          pos_hbm, rel_hbm,
          x_v, dpost_v, dpara_v, ppost_v, ppara_v, ct_local, pt_local,
          codes_v, pcodes_v,
          rel_bufs0, rel_bufs1, pos_buf,
          ct_sh, pt_sh,
          sem_g, sem_r0, sem_r1, sem_pg, sem_po):
    cid = lax.axis_index("c")
    sid = lax.axis_index("s")
    wid = sid * 2 + cid
    b0 = wid * NB_W

    @pl.when(sid == 0)
    def _build_tables():
        pltpu.sync_copy(dpost_hbm, dpost_v)
        pltpu.sync_copy(dpara_hbm, dpara_v)
        for c in range(64):
            ct_local[c, pl.ds(0, 16)] = dpara_v[c >> 2, pl.ds(0, 16)]
            ct_local[c, pl.ds(16, 16)] = dpost_v[c & 3, pl.ds(0, 16)]
        pltpu.sync_copy(ct_local, ct_sh)
        pltpu.sync_copy(ppost_hbm, ppost_v)
        pltpu.sync_copy(ppara_hbm, ppara_v)

        def pt_row(r, carry):
            pt_local[r, pl.ds(0, 16)] = ppost_v[r >> 4, pl.ds(0, 16)]
            pt_local[r, pl.ds(16, 16)] = ppara_v[r & 15, pl.ds(0, 16)]
            return carry

        lax.fori_loop(0, 256, pt_row, 0)
        pltpu.sync_copy(pt_local, pt_sh)

    pltpu.sync_copy(x_hbm.at[pl.ds(b0 * 2 * S, NB_W * 2 * S)], x_v)
    plsc.subcore_barrier()

    rel_bufs = (rel_bufs0, rel_bufs1)
    rel_sems = (sem_r0, sem_r1)

    def _codes_batch(bl, cbase, pbase):
        xoff = bl * 2 * S
        xj0 = [x_v[pl.ds(xoff + c, LANES)] for c in J_CHUNKS]
        xj1 = [x_v[pl.ds(xoff + S + c, LANES)] for c in J_CHUNKS]

        for ci, c in enumerate(J_CHUNKS):
            i0 = jnp.minimum(jnp.abs(xj0[ci]), 15)
            i1 = jnp.minimum(jnp.abs(xj1[ci]), 15)
            pcodes_v[pl.ds(pbase + c, LANES)] = (i0 << 4) | i1

        def i_row(il, carry):
            xi0 = plsc.load_gather(x_v, [_splat(xoff + il)])
            xi1 = plsc.load_gather(x_v, [_splat(xoff + S + il)])
            obase = cbase + il * S
            for ci, c in enumerate(J_CHUNKS):
                a = jnp.minimum(jnp.abs(xj0[ci] - xi0), 15)
                p = jnp.minimum(jnp.abs(xj1[ci] - xi1), 3)
                codes_v[pl.ds(obase + c, LANES)] = (a << 2) | p
            return carry

        lax.fori_loop(0, S, i_row, 0)

    def group(g, carry):
        for pr in range(2):
            pb = g * 4 + pr * 2
            pair_row0 = (b0 + pb) * ROWS_B
            _codes_batch(pb, 0, (pr * 2) * S)
            _codes_batch(pb + 1, ROWS_B, (pr * 2 + 1) * S)
            for kc, (off, cnt) in enumerate(REL_CHUNKS):
                sl = kc % 2
                buf = rel_bufs[sl].at[pl.ds(0, cnt)]
                cnt_prev = REL_CHUNKS[kc - 2][1] if kc >= 2 else (
                    1248 if sl == 0 else 1256)
                prev_wait = pltpu.make_async_copy(
                    rel_bufs[sl].at[pl.ds(0, cnt_prev)],
                    rel_hbm.at[pl.ds(pl.multiple_of(pair_row0, 8), cnt_prev)],
                    rel_sems[sl])
                if kc >= 2 or pr == 1:
                    prev_wait.wait()
                else:
                    @pl.when(g > 0)
                    def _w():
                        prev_wait.wait()
                gchunks = [(o2, min(96, cnt - o2)) for o2 in range(0, cnt, 96)]
                gcopies = [
                    pltpu.make_async_copy(
                        ct_sh.at[codes_v.at[pl.ds(off + o2, c2)]],
                        rel_bufs[sl].at[pl.ds(o2, c2)], sem_g)
                    for o2, c2 in gchunks]
                for gc in gcopies:
                    gc.start()
                for gc in gcopies:
                    gc.wait()
                pltpu.make_async_copy(
                    buf,
                    rel_hbm.at[pl.ds(pl.multiple_of(pair_row0 + off, 8), cnt)],
                    rel_sems[sl]).start()

        pos_row0 = (b0 + g * 4) * S
        pos_out = pos_hbm.at[pl.ds(pl.multiple_of(pos_row0, 8), 4 * S)]
        pos_wait = pltpu.make_async_copy(pos_buf, pos_out, sem_po)

        @pl.when(g > 0)
        def _wp():
            pos_wait.wait()

        pchunks = [(0, 96), (96, 96), (192, 8)]
        pcopies = [
            pltpu.make_async_copy(
                pt_sh.at[pcodes_v.at[pl.ds(o2, c2)]],
                pos_buf.at[pl.ds(o2, c2)], sem_pg)
            for o2, c2 in pchunks]
        for pc in pcopies:
            pc.start()
        for pc in pcopies:
            pc.wait()
        pltpu.make_async_copy(pos_buf, pos_out, sem_po).start()
        return carry

    lax.fori_loop(0, NB_W // 4, group, 0)

    last_row0 = (b0 + NB_W - 2) * ROWS_B
    pltpu.make_async_copy(
        rel_bufs[0].at[pl.ds(0, 1248)],
        rel_hbm.at[pl.ds(pl.multiple_of(last_row0, 8), 1248)],
        sem_r0).wait()
    pltpu.make_async_copy(
        rel_bufs[1].at[pl.ds(0, 1256)],
        rel_hbm.at[pl.ds(pl.multiple_of(last_row0 + 3744, 8), 1256)],
        sem_r1).wait()
    pltpu.make_async_copy(
        pos_buf,
        pos_hbm.at[pl.ds(pl.multiple_of((b0 + NB_W - 4) * S, 8), 4 * S)],
        sem_po).wait()


@jax.jit
def _sc_position_layer(x, ppost, ppara, dpost, dpara):
    mesh = plsc.VectorSubcoreMesh(core_axis_name="c", subcore_axis_name="s")
    f = pl.kernel(
        _body,
        out_type=(jax.ShapeDtypeStruct((B * S, 32), jnp.float32),
                  jax.ShapeDtypeStruct((B * ROWS_B, 32), jnp.float32)),
        mesh=mesh,
        scratch_types=[
            pltpu.VMEM((NB_W * 2 * S,), jnp.int32),
            pltpu.VMEM((4, 16), jnp.float32),
            pltpu.VMEM((16, 16), jnp.float32),
            pltpu.VMEM((16, 16), jnp.float32),
            pltpu.VMEM((16, 16), jnp.float32),
            pltpu.VMEM((64, 32), jnp.float32),
            pltpu.VMEM((256, 32), jnp.float32),
            pltpu.VMEM((2 * ROWS_B + 8,), jnp.int32),
            pltpu.VMEM((208,), jnp.int32),
            pltpu.VMEM((CMAX, 32), jnp.float32),
            pltpu.VMEM((CMAX, 32), jnp.float32),
            pltpu.VMEM((4 * S, 32), jnp.float32),
            pltpu.VMEM_SHARED((64, 32), jnp.float32),
            pltpu.VMEM_SHARED((256, 32), jnp.float32),
            pltpu.SemaphoreType.DMA,
            pltpu.SemaphoreType.DMA,
            pltpu.SemaphoreType.DMA,
            pltpu.SemaphoreType.DMA,
            pltpu.SemaphoreType.DMA,
        ],
        compiler_params=pltpu.CompilerParams(needs_layout_passes=False,
                                             use_tc_tiling_on_sc=False),
    )
    return f(x, ppost, ppara, dpost, dpara)


def kernel(x_position_info, pos_post_emb, pos_para_emb, dist_post_emb, dist_para_emb):
    x = x_position_info.astype(jnp.int32).transpose(0, 2, 1).reshape(B * 2 * S)
    pos_flat, rel_flat = _sc_position_layer(
        x, pos_post_emb, pos_para_emb, dist_post_emb, dist_para_emb)
    return (pos_flat.reshape(B, S, 32), rel_flat.reshape(B, S, S, 32))
